# baseline (speedup 1.0000x reference)
import functools

import jax
import jax.numpy as jnp
from jax import lax
from jax.experimental import pallas as pl

N_SAMPLES = 256
IOU_THRESH = 0.5

_B, _N, _G, _C = 8, 8192, 128, 81
_NR, _NL = 64, 128
_CHUNK = 1024
_NCHUNK = _N // _CHUNK


def _prep_kernel(cx, cy, w, h, ang, x1o, y1o, x2o, y2o, ao):
    f32 = jnp.float32
    t = ang[0] * f32(jnp.pi / 180.0)
    act = jnp.abs(jnp.cos(t))
    ast = jnp.abs(jnp.sin(t))
    hw = w[0] * f32(0.5)
    hh = h[0] * f32(0.5)
    dx = hw * act + hh * ast
    dy = hw * ast + hh * act
    x1o[0] = cx[0] - dx
    y1o[0] = cy[0] - dy
    x2o[0] = cx[0] + dx
    y2o[0] = cy[0] + dy
    ao[0] = (dx + dx) * (dy + dy)


def _kernel_a(ax1, ay1, ax2, ay2, aarea, cls_pred,
              tx1, ty1, tx2, ty2, bt0, bt1, bt2, bt3, bt4, gcls,
              score_o, ce_pos_o, ce_bg_o, t0o, t1o, t2o, t3o, t4o):
    f32 = jnp.float32
    rx1 = ax1[0]
    ry1 = ay1[0]
    rx2 = ax2[0]
    ry2 = ay2[0]
    area_a = aarea[0]

    bx1 = tx1[0]
    by1 = ty1[0]
    bx2 = tx2[0]
    by2 = ty2[0]
    area_b = (bx2 - bx1) * (by2 - by1)

    iw = jnp.clip(jnp.minimum(rx2, bx2) - jnp.maximum(rx1, bx1), 0.0, None)
    ih = jnp.clip(jnp.minimum(ry2, by2) - jnp.maximum(ry1, by1), 0.0, None)
    inter = iw * ih
    union = jnp.clip(area_a + area_b - inter, 1e-8, None)
    iou = inter / union

    max_iou = jnp.max(iou, axis=1, keepdims=True)
    g_iota = lax.broadcasted_iota(jnp.int32, (_CHUNK, _G), 1)
    amax = jnp.min(jnp.where(iou == max_iou, g_iota, _G), axis=1,
                   keepdims=True)
    onehot = amax == g_iota

    pos = max_iou >= f32(IOU_THRESH)
    score_o[0] = jnp.where(pos, f32(2.0) + max_iou, max_iou)

    zero = jnp.zeros((), f32)
    cls_t = jnp.sum(jnp.where(onehot, gcls[0], 0), axis=1,
                    keepdims=True)
    for bt, to in ((bt0, t0o), (bt1, t1o), (bt2, t2o), (bt3, t3o), (bt4, t4o)):
        to[0] = jnp.sum(jnp.where(onehot, bt[0], zero), axis=1, keepdims=True)

    logits = cls_pred[0]
    m = jnp.max(logits, axis=1, keepdims=True)
    s = jnp.sum(jnp.exp(logits - m), axis=1, keepdims=True)
    lse = m + jnp.log(s)
    c_iota = lax.broadcasted_iota(jnp.int32, (_CHUNK, _C), 1)
    logit_cls = jnp.sum(jnp.where(c_iota == cls_t, logits, zero), axis=1,
                        keepdims=True)
    logit_bg = logits[:, _C - 1:_C]
    ce_pos_o[0] = lse - logit_cls
    ce_bg_o[0] = lse - logit_bg


def _kernel_b(score, ce_pos, ce_bg, t0, t1, t2, t3, t4,
              bp0, bp1, bp2, bp3, bp4, out):
    f32 = jnp.float32
    i32 = jnp.int32
    s = score[...]
    s_bits = lax.bitcast_convert_type(s, i32)

    def count3(mask):
        return jnp.sum(jnp.sum(mask.astype(i32), axis=2, keepdims=True),
                       axis=1, keepdims=True)

    k = jnp.full((_B, 1, 1), N_SAMPLES, i32)

    def t_body(i, u):
        cand = u | jnp.left_shift(jnp.ones((), i32), 30 - i)
        return jnp.where(count3(s_bits >= cand) >= k, cand, u)

    t = lax.fori_loop(0, 31, t_body, jnp.zeros((_B, 1, 1), i32))
    n_gt = count3(s_bits > t)
    k_tie = k - n_gt
    tie = s_bits == t
    idx = (lax.broadcasted_iota(i32, (_B, _NR, _NL), 1) * _NL
           + lax.broadcasted_iota(i32, (_B, _NR, _NL), 2))

    def q_body(i, q):
        cand = q | jnp.left_shift(jnp.ones((), i32), 12 - i)
        return jnp.where(count3(tie & (idx < cand)) < k_tie, cand, q)

    q = lax.fori_loop(0, 13, q_body, jnp.zeros((_B, 1, 1), i32))
    sel = (s_bits > t) | (tie & (idx <= q))
    pos = s >= f32(2.0)
    zero = jnp.zeros((), f32)

    r = jnp.zeros((_B, _NR, _NL), f32)
    for bp, tc in ((bp0, t0), (bp1, t1), (bp2, t2), (bp3, t3), (bp4, t4)):
        d = bp[...] - tc[...]
        ad = jnp.abs(d)
        r = r + jnp.where(ad < f32(1.0), f32(0.5) * d * d, ad - f32(0.5))

    def sum3(x):
        return jnp.sum(jnp.sum(x, axis=2, keepdims=True), axis=1,
                       keepdims=True)

    cls_l = sum3(jnp.where(sel, jnp.where(pos, ce_pos[...], ce_bg[...]), zero))
    reg_l = sum3(jnp.where(sel & pos, r, zero))
    out[:, 0:1, :] = cls_l[:, :, 0:1]
    out[:, 1:2, :] = reg_l[:, :, 0:1]


@functools.partial(jax.jit, static_argnames=("interpret",))
def kernel(box_pred, class_pred, rois, gt_boxes, gt_classes, interpret=False):
    f32 = jnp.float32
    B = box_pred.shape[0]
    c = gt_boxes.mean(axis=-2)
    e1 = gt_boxes[..., 1, :] - gt_boxes[..., 0, :]
    e2 = gt_boxes[..., 2, :] - gt_boxes[..., 1, :]
    gw = jnp.linalg.norm(e1, axis=-1)
    gh = jnp.linalg.norm(e2, axis=-1)
    gang = jnp.degrees(jnp.arctan2(e1[..., 1], e1[..., 0]))
    bt = (c[..., 0], c[..., 1], gw, gh, gang)
    mn = gt_boxes.min(axis=-2)
    mx = gt_boxes.max(axis=-2)
    thbb = (mn[..., 0], mn[..., 1], mx[..., 0], mx[..., 1])

    def g2(x):
        return x.astype(f32).reshape(B, 1, _G)

    roi_c = [rois[..., i].astype(f32).reshape(B, _NR, _NL) for i in range(5)]
    img_spec = pl.BlockSpec((1, _NR, _NL), lambda b: (b, 0, 0))
    hbb = pl.pallas_call(
        _prep_kernel,
        grid=(B,),
        in_specs=[img_spec] * 5,
        out_specs=[img_spec] * 5,
        out_shape=[jax.ShapeDtypeStruct((B, _NR, _NL), f32)] * 5,
        interpret=interpret,
    )(*roi_c)
    hbb_n1 = [x.reshape(B, _N, 1) for x in hbb]

    gt_in = [g2(x) for x in thbb] + [g2(x) for x in bt]
    gcls = gt_classes.astype(jnp.int32).reshape(B, 1, _G)

    n1_spec = pl.BlockSpec((1, _CHUNK, 1), lambda b, c_: (b, c_, 0))
    g_spec = pl.BlockSpec((1, 1, _G), lambda b, c_: (b, 0, 0))
    cp_spec = pl.BlockSpec((1, _CHUNK, _C), lambda b, c_: (b, c_, 0))
    outs = pl.pallas_call(
        _kernel_a,
        grid=(B, _NCHUNK),
        in_specs=[n1_spec] * 5 + [cp_spec] + [g_spec] * 10,
        out_specs=[n1_spec] * 8,
        out_shape=[jax.ShapeDtypeStruct((B, _N, 1), f32)] * 8,
        interpret=interpret,
    )(*hbb_n1, class_pred, *gt_in, gcls)
    outs = [x.reshape(B, _NR, _NL) for x in outs]
    bp_c = [box_pred[..., i].astype(f32).reshape(B, _NR, _NL)
            for i in range(5)]

    per_img = pl.pallas_call(
        _kernel_b,
        out_shape=jax.ShapeDtypeStruct((B, 2, 1), f32),
        interpret=interpret,
    )(*outs, *bp_c)

    cls_loss = per_img[:, 0, 0].sum() / B
    regr_loss = per_img[:, 1, 0].sum() / B
    return (cls_loss + regr_loss, cls_loss, regr_loss)


# speedup vs baseline: 3.8996x; 3.8996x over previous
import functools

import jax
import jax.numpy as jnp
from jax import lax
from jax.experimental import pallas as pl

N_SAMPLES = 256
IOU_THRESH = 0.5

_B, _N, _G, _C = 8, 8192, 128, 81
_NR, _NL = 64, 128
_CHUNK = 1024
_NCHUNK = _N // _CHUNK


def _kernel_a(rr, cpt, tx1, ty1, tx2, ty2, bt0, bt1, bt2, bt3, bt4, gcls,
              out):
    f32 = jnp.float32
    r = rr[0]
    cx = r[0:1, :]
    cy = r[1:2, :]
    t = r[4:5, :] * f32(jnp.pi / 180.0)
    act = jnp.abs(jnp.cos(t))
    ast = jnp.abs(jnp.sin(t))
    hw = r[2:3, :] * f32(0.5)
    hh = r[3:4, :] * f32(0.5)
    dx = hw * act + hh * ast
    dy = hw * ast + hh * act
    rx1 = cx - dx
    ry1 = cy - dy
    rx2 = cx + dx
    ry2 = cy + dy
    area_a = (rx2 - rx1) * (ry2 - ry1)

    bx1 = tx1[0]
    by1 = ty1[0]
    bx2 = tx2[0]
    by2 = ty2[0]
    area_b = (bx2 - bx1) * (by2 - by1)

    iw = jnp.clip(jnp.minimum(rx2, bx2) - jnp.maximum(rx1, bx1), 0.0, None)
    ih = jnp.clip(jnp.minimum(ry2, by2) - jnp.maximum(ry1, by1), 0.0, None)
    inter = iw * ih
    union = jnp.clip(area_a + area_b - inter, 1e-8, None)
    iou = inter / union

    max_iou = jnp.max(iou, axis=0, keepdims=True)
    g_iota = lax.broadcasted_iota(jnp.int32, (_G, _CHUNK), 0)
    amax = jnp.min(jnp.where(iou == max_iou, g_iota, _G), axis=0,
                   keepdims=True)
    onehot = g_iota == amax

    pos = max_iou >= f32(IOU_THRESH)
    score = jnp.where(pos, f32(2.0) + max_iou, max_iou)

    zero = jnp.zeros((), f32)
    cls_t = jnp.sum(jnp.where(onehot, gcls[0], 0), axis=0,
                    keepdims=True)
    tg = [jnp.sum(jnp.where(onehot, bt[0], zero), axis=0, keepdims=True)
          for bt in (bt0, bt1, bt2, bt3, bt4)]

    lg = cpt[0]
    m = jnp.max(lg, axis=0, keepdims=True)
    s = jnp.sum(jnp.exp(lg - m), axis=0, keepdims=True)
    lse = m + jnp.log(s)
    c_iota = lax.broadcasted_iota(jnp.int32, (_C, _CHUNK), 0)
    logit_cls = jnp.sum(jnp.where(c_iota == cls_t, lg, zero), axis=0,
                        keepdims=True)
    logit_bg = lg[_C - 1:_C, :]
    out[0] = jnp.concatenate(
        [score, lse - logit_cls, lse - logit_bg] + tg, axis=0)


def _kernel_b(score, ce_pos, ce_bg, t0, t1, t2, t3, t4,
              bp0, bp1, bp2, bp3, bp4, out):
    f32 = jnp.float32
    i32 = jnp.int32
    s = score[...]
    s_bits = lax.bitcast_convert_type(s, i32)

    def count3(mask):
        return jnp.sum(jnp.sum(mask.astype(i32), axis=2, keepdims=True),
                       axis=1, keepdims=True)

    k = jnp.full((_B, 1, 1), N_SAMPLES, i32)

    def t_body(i, u):
        cand = u | jnp.left_shift(jnp.ones((), i32), 30 - i)
        return jnp.where(count3(s_bits >= cand) >= k, cand, u)

    t = lax.fori_loop(0, 31, t_body, jnp.zeros((_B, 1, 1), i32))
    n_gt = count3(s_bits > t)
    k_tie = k - n_gt
    tie = s_bits == t
    idx = (lax.broadcasted_iota(i32, (_B, _NR, _NL), 1) * _NL
           + lax.broadcasted_iota(i32, (_B, _NR, _NL), 2))

    def q_body(i, q):
        cand = q | jnp.left_shift(jnp.ones((), i32), 12 - i)
        return jnp.where(count3(tie & (idx < cand)) < k_tie, cand, q)

    q = lax.fori_loop(0, 13, q_body, jnp.zeros((_B, 1, 1), i32))
    sel = (s_bits > t) | (tie & (idx <= q))
    pos = s >= f32(2.0)
    zero = jnp.zeros((), f32)

    r = jnp.zeros((_B, _NR, _NL), f32)
    for bp, tc in ((bp0, t0), (bp1, t1), (bp2, t2), (bp3, t3), (bp4, t4)):
        d = bp[...] - tc[...]
        ad = jnp.abs(d)
        r = r + jnp.where(ad < f32(1.0), f32(0.5) * d * d, ad - f32(0.5))

    def sum3(x):
        return jnp.sum(jnp.sum(x, axis=2, keepdims=True), axis=1,
                       keepdims=True)

    cls_l = sum3(jnp.where(sel, jnp.where(pos, ce_pos[...], ce_bg[...]), zero))
    reg_l = sum3(jnp.where(sel & pos, r, zero))
    out[:, 0:1, :] = cls_l[:, :, 0:1]
    out[:, 1:2, :] = reg_l[:, :, 0:1]


@functools.partial(jax.jit, static_argnames=("interpret",))
def kernel(box_pred, class_pred, rois, gt_boxes, gt_classes, interpret=False):
    f32 = jnp.float32
    B = box_pred.shape[0]
    c = gt_boxes.mean(axis=-2)
    e1 = gt_boxes[..., 1, :] - gt_boxes[..., 0, :]
    e2 = gt_boxes[..., 2, :] - gt_boxes[..., 1, :]
    gw = jnp.linalg.norm(e1, axis=-1)
    gh = jnp.linalg.norm(e2, axis=-1)
    gang = jnp.degrees(jnp.arctan2(e1[..., 1], e1[..., 0]))
    bt = (c[..., 0], c[..., 1], gw, gh, gang)
    mn = gt_boxes.min(axis=-2)
    mx = gt_boxes.max(axis=-2)
    thbb = (mn[..., 0], mn[..., 1], mx[..., 0], mx[..., 1])

    def g2(x):
        return x.astype(f32).reshape(B, _G, 1)

    rr = jnp.swapaxes(rois, 1, 2).astype(f32)
    cpt = jnp.swapaxes(class_pred, 1, 2).astype(f32)
    gt_in = [g2(x) for x in thbb] + [g2(x) for x in bt]
    gcls = gt_classes.astype(jnp.int32).reshape(B, _G, 1)

    rr_spec = pl.BlockSpec((1, 5, _CHUNK), lambda b, c_: (b, 0, c_))
    cp_spec = pl.BlockSpec((1, _C, _CHUNK), lambda b, c_: (b, 0, c_))
    g_spec = pl.BlockSpec((1, _G, 1), lambda b, c_: (b, 0, 0))
    out_spec = pl.BlockSpec((1, 8, _CHUNK), lambda b, c_: (b, 0, c_))
    oa = pl.pallas_call(
        _kernel_a,
        grid=(B, _NCHUNK),
        in_specs=[rr_spec, cp_spec] + [g_spec] * 10,
        out_specs=out_spec,
        out_shape=jax.ShapeDtypeStruct((B, 8, _N), f32),
        interpret=interpret,
    )(rr, cpt, *gt_in, gcls)

    per_roi = [oa[:, i, :].reshape(B, _NR, _NL) for i in range(8)]
    bp_c = [box_pred[..., i].astype(f32).reshape(B, _NR, _NL)
            for i in range(5)]

    per_img = pl.pallas_call(
        _kernel_b,
        out_shape=jax.ShapeDtypeStruct((B, 2, 1), f32),
        interpret=interpret,
    )(*per_roi, *bp_c)

    cls_loss = per_img[:, 0, 0].sum() / B
    regr_loss = per_img[:, 1, 0].sum() / B
    return (cls_loss + regr_loss, cls_loss, regr_loss)


# speedup vs baseline: 4.3206x; 1.1080x over previous
import functools

import jax
import jax.numpy as jnp
from jax import lax
from jax.experimental import pallas as pl

N_SAMPLES = 256
IOU_THRESH = 0.5

_B, _N, _G, _C = 8, 8192, 128, 81
_NR, _NL = 64, 128
_CHUNK = 1024
_NCHUNK = _N // _CHUNK


def _kernel_a(rr, cpt, tx1, ty1, tx2, ty2, gtm, out):
    f32 = jnp.float32
    r = rr[0]
    cx = r[0:1, :]
    cy = r[1:2, :]
    t = r[4:5, :] * f32(jnp.pi / 180.0)
    act = jnp.abs(jnp.cos(t))
    ast = jnp.abs(jnp.sin(t))
    hw = r[2:3, :] * f32(0.5)
    hh = r[3:4, :] * f32(0.5)
    dx = hw * act + hh * ast
    dy = hw * ast + hh * act
    rx1 = cx - dx
    ry1 = cy - dy
    rx2 = cx + dx
    ry2 = cy + dy
    area_a = (rx2 - rx1) * (ry2 - ry1)

    bx1 = tx1[0]
    by1 = ty1[0]
    bx2 = tx2[0]
    by2 = ty2[0]
    area_b = (bx2 - bx1) * (by2 - by1)

    iw = jnp.clip(jnp.minimum(rx2, bx2) - jnp.maximum(rx1, bx1), 0.0, None)
    ih = jnp.clip(jnp.minimum(ry2, by2) - jnp.maximum(ry1, by1), 0.0, None)
    inter = iw * ih
    union = jnp.clip(area_a + area_b - inter, 1e-8, None)
    iou = inter / union

    max_iou = jnp.max(iou, axis=0, keepdims=True)
    g_iota = lax.broadcasted_iota(jnp.int32, (_G, _CHUNK), 0)
    amax = jnp.min(jnp.where(iou == max_iou, g_iota, _G), axis=0,
                   keepdims=True)
    onehot = g_iota == amax

    pos = max_iou >= f32(IOU_THRESH)
    score = jnp.where(pos, f32(2.0) + max_iou, max_iou)

    zero = jnp.zeros((), f32)
    gath = lax.dot_general(gtm[0], onehot.astype(f32),
                           (((1,), (0,)), ((), ())),
                           preferred_element_type=f32)
    cls_t = gath[0:1, :].astype(jnp.int32)
    tg = [gath[i:i + 1, :] for i in range(1, 6)]

    lg = cpt[0]
    m = jnp.max(lg, axis=0, keepdims=True)
    s = jnp.sum(jnp.exp(lg - m), axis=0, keepdims=True)
    lse = m + jnp.log(s)
    c_iota = lax.broadcasted_iota(jnp.int32, (_C, _CHUNK), 0)
    logit_cls = jnp.sum(jnp.where(c_iota == cls_t, lg, zero), axis=0,
                        keepdims=True)
    logit_bg = lg[_C - 1:_C, :]
    out[0] = jnp.concatenate(
        [score, lse - logit_cls, lse - logit_bg] + tg, axis=0)


def _kernel_b(score, ce_pos, ce_bg, t0, t1, t2, t3, t4,
              bp0, bp1, bp2, bp3, bp4, out):
    f32 = jnp.float32
    i32 = jnp.int32
    s = score[...]
    s_bits = lax.bitcast_convert_type(s, i32)

    def count3(mask):
        return jnp.sum(jnp.sum(mask.astype(i32), axis=2, keepdims=True),
                       axis=1, keepdims=True)

    k = jnp.full((_B, 1, 1), N_SAMPLES, i32)

    def t_body(i, u):
        cand = u | jnp.left_shift(jnp.ones((), i32), 30 - i)
        return jnp.where(count3(s_bits >= cand) >= k, cand, u)

    t = lax.fori_loop(0, 31, t_body, jnp.zeros((_B, 1, 1), i32))
    n_gt = count3(s_bits > t)
    k_tie = k - n_gt
    tie = s_bits == t
    idx = (lax.broadcasted_iota(i32, (_B, _NR, _NL), 1) * _NL
           + lax.broadcasted_iota(i32, (_B, _NR, _NL), 2))

    def q_body(i, q):
        cand = q | jnp.left_shift(jnp.ones((), i32), 12 - i)
        return jnp.where(count3(tie & (idx < cand)) < k_tie, cand, q)

    q = lax.fori_loop(0, 13, q_body, jnp.zeros((_B, 1, 1), i32))
    sel = (s_bits > t) | (tie & (idx <= q))
    pos = s >= f32(2.0)
    zero = jnp.zeros((), f32)

    r = jnp.zeros((_B, _NR, _NL), f32)
    for bp, tc in ((bp0, t0), (bp1, t1), (bp2, t2), (bp3, t3), (bp4, t4)):
        d = bp[...] - tc[...]
        ad = jnp.abs(d)
        r = r + jnp.where(ad < f32(1.0), f32(0.5) * d * d, ad - f32(0.5))

    def sum3(x):
        return jnp.sum(jnp.sum(x, axis=2, keepdims=True), axis=1,
                       keepdims=True)

    cls_l = sum3(jnp.where(sel, jnp.where(pos, ce_pos[...], ce_bg[...]), zero))
    reg_l = sum3(jnp.where(sel & pos, r, zero))
    out[:, 0:1, :] = cls_l[:, :, 0:1]
    out[:, 1:2, :] = reg_l[:, :, 0:1]


@functools.partial(jax.jit, static_argnames=("interpret",))
def kernel(box_pred, class_pred, rois, gt_boxes, gt_classes, interpret=False):
    f32 = jnp.float32
    B = box_pred.shape[0]
    c = gt_boxes.mean(axis=-2)
    e1 = gt_boxes[..., 1, :] - gt_boxes[..., 0, :]
    e2 = gt_boxes[..., 2, :] - gt_boxes[..., 1, :]
    gw = jnp.linalg.norm(e1, axis=-1)
    gh = jnp.linalg.norm(e2, axis=-1)
    gang = jnp.degrees(jnp.arctan2(e1[..., 1], e1[..., 0]))
    bt = (c[..., 0], c[..., 1], gw, gh, gang)
    mn = gt_boxes.min(axis=-2)
    mx = gt_boxes.max(axis=-2)
    thbb = (mn[..., 0], mn[..., 1], mx[..., 0], mx[..., 1])

    def g2(x):
        return x.astype(f32).reshape(B, _G, 1)

    rr = jnp.swapaxes(rois, 1, 2).astype(f32)
    cpt = jnp.swapaxes(class_pred, 1, 2).astype(f32)
    gt_in = [g2(x) for x in thbb]
    gtm = jnp.stack([gt_classes.astype(f32)] + [x.astype(f32) for x in bt]
                    + [jnp.zeros((B, _G), f32)] * 2, axis=1)

    rr_spec = pl.BlockSpec((1, 5, _CHUNK), lambda b, c_: (b, 0, c_))
    cp_spec = pl.BlockSpec((1, _C, _CHUNK), lambda b, c_: (b, 0, c_))
    g_spec = pl.BlockSpec((1, _G, 1), lambda b, c_: (b, 0, 0))
    gtm_spec = pl.BlockSpec((1, 8, _G), lambda b, c_: (b, 0, 0))
    out_spec = pl.BlockSpec((1, 8, _CHUNK), lambda b, c_: (b, 0, c_))
    oa = pl.pallas_call(
        _kernel_a,
        grid=(B, _NCHUNK),
        in_specs=[rr_spec, cp_spec] + [g_spec] * 4 + [gtm_spec],
        out_specs=out_spec,
        out_shape=jax.ShapeDtypeStruct((B, 8, _N), f32),
        interpret=interpret,
    )(rr, cpt, *gt_in, gtm)

    per_roi = [oa[:, i, :].reshape(B, _NR, _NL) for i in range(8)]
    bp_c = [box_pred[..., i].astype(f32).reshape(B, _NR, _NL)
            for i in range(5)]

    per_img = pl.pallas_call(
        _kernel_b,
        out_shape=jax.ShapeDtypeStruct((B, 2, 1), f32),
        interpret=interpret,
    )(*per_roi, *bp_c)

    cls_loss = per_img[:, 0, 0].sum() / B
    regr_loss = per_img[:, 1, 0].sum() / B
    return (cls_loss + regr_loss, cls_loss, regr_loss)
